# Initial kernel scaffold; baseline (speedup 1.0000x reference)
import jax
import jax.numpy as jnp
from jax.experimental import pallas as pl


def kernel(street_embedding, edge_index, y, train_mask, W1, a1_src, a1_dst, b1, W2, a2_src, a2_dst, b2):
    raise NotImplementedError("write your pallas kernel here")


# speedup vs baseline: 1.0675x; 1.0675x over previous
import jax
import jax.numpy as jnp
from jax.experimental import pallas as pl

N = 10000
D = 128
H1, C1 = 8, 64
H2, C2 = 10, 10


def _gat_conv(x, edge_index, W, a_src, a_dst, bias, heads, out_ch, concat):
    n = x.shape[0]
    loops = jnp.arange(n, dtype=edge_index.dtype)
    src = jnp.concatenate([edge_index[0], loops])
    dst = jnp.concatenate([edge_index[1], loops])
    h = (x @ W).reshape(n, heads, out_ch)
    alpha_s = (h * a_src).sum(-1)
    alpha_d = (h * a_dst).sum(-1)
    e = alpha_s[src] + alpha_d[dst]
    e = jax.nn.leaky_relu(e, 0.2)
    w = jnp.exp(e)
    denom = jax.ops.segment_sum(w, dst, num_segments=n)
    msg = h[src] * w[..., None]
    out = jax.ops.segment_sum(msg, dst, num_segments=n)
    out = out / (denom[..., None] + 1e-16)
    if concat:
        out = out.reshape(n, heads * out_ch)
    else:
        out = out.mean(axis=1)
    return out + bias


def _nll_kernel(h2_ref, y_ref, out_ref):
    h2 = h2_ref[...]
    m = jnp.max(h2, axis=-1, keepdims=True)
    lse = jnp.log(jnp.sum(jnp.exp(h2 - m), axis=-1, keepdims=True)) + m
    logp = h2 - lse
    y = y_ref[...]
    onehot = (jax.lax.broadcasted_iota(jnp.int32, h2.shape, 1) == y[:, None])
    out_ref[...] = -jnp.sum(jnp.where(onehot, logp, 0.0), axis=-1)


def kernel(street_embedding, edge_index, y, train_mask, W1, a1_src, a1_dst, b1, W2, a2_src, a2_dst, b2):
    h1 = _gat_conv(street_embedding, edge_index, W1, a1_src, a1_dst, b1, H1, C1, True)
    h = jax.nn.elu(h1)
    h2 = _gat_conv(h, edge_index, W2, a2_src, a2_dst, b2, H2, C2, False)
    nll = pl.pallas_call(
        _nll_kernel,
        out_shape=jax.ShapeDtypeStruct((N,), jnp.float32),
    )(h2, y.astype(jnp.int32))
    m = train_mask.astype(jnp.float32)
    loss_su = (nll * m).sum() / jnp.maximum(m.sum(), 1.0)
    return (loss_su, h1, h2)


# speedup vs baseline: 17.0617x; 15.9827x over previous
import functools

import jax
import jax.numpy as jnp
from jax import lax
from jax.experimental import pallas as pl
from jax.experimental.pallas import tpu as pltpu
from jax.experimental.pallas import tpu_sc as plsc

N = 10000
E = 320000
D = 128
H1, C1 = 8, 64
H2, C2 = 10, 10

NC, NS = 2, 16
NT = NC * NS
EPT = E // NT
CHUNK = 80
NCHUNK = EPT // CHUNK
NPAD = 10240
RPT = NPAD // NS
ZROW = 128

_mesh = plsc.VectorSubcoreMesh(core_axis_name="c", subcore_axis_name="s")


def _splat(v):
    return jnp.full((16,), v, jnp.int32)


_LANE = None


@functools.partial(
    pl.kernel,
    mesh=_mesh,
    compiler_params=pltpu.CompilerParams(needs_layout_passes=False),
    out_type=(
        jax.ShapeDtypeStruct((E, 16), jnp.float32),
        jax.ShapeDtypeStruct((NC * NPAD, 128), jnp.float32),
    ),
    scratch_types=(
        pltpu.VMEM((CHUNK,), jnp.int32),
        pltpu.VMEM((CHUNK,), jnp.int32),
        pltpu.VMEM((CHUNK, 128), jnp.float32),
        pltpu.VMEM((CHUNK, 128), jnp.float32),
        pltpu.VMEM((CHUNK, 16), jnp.float32),
        pltpu.VMEM((CHUNK, 128), jnp.float32),
        pltpu.VMEM_SHARED((NPAD, 128), jnp.float32),
        pltpu.SemaphoreType.DMA,
    ),
)
def _w_kernel(asad_hbm, src_hbm, dst_hbm, zpad_hbm, zstg_hbm,
              w_hbm, den_hbm,
              sidx, didx, abuf, bbuf, wbuf, wpad, den_acc, sem):
    cid = lax.axis_index("c")
    sid = lax.axis_index("s")
    tile_base = (cid * NS + sid) * EPT
    row0 = sid * RPT

    pltpu.sync_copy(zpad_hbm, wpad)
    for i in range(RPT // ZROW):
        pltpu.sync_copy(zstg_hbm, den_acc.at[pl.ds(row0 + i * ZROW, ZROW)])
    plsc.subcore_barrier()

    def _chunk(ci, carry):
        base = tile_base + ci * CHUNK
        pltpu.sync_copy(src_hbm.at[pl.ds(base, CHUNK)], sidx)
        pltpu.sync_copy(dst_hbm.at[pl.ds(base, CHUNK)], didx)
        pltpu.async_copy(asad_hbm.at[sidx], abuf, sem).wait()
        pltpu.async_copy(asad_hbm.at[didx], bbuf, sem).wait()

        def _edge(e, c2):
            x = abuf[e, pl.ds(0, 16)] + bbuf[e, pl.ds(16, 16)]
            x = jnp.where(x >= 0.0, x, 0.2 * x)
            w16 = jnp.exp(x)
            wbuf[e, pl.ds(0, 16)] = w16
            wpad[e, pl.ds(0, 16)] = w16
            return c2
        lax.fori_loop(0, CHUNK, _edge, 0)

        pltpu.sync_copy(wpad, den_acc.at[didx], add=True)
        pltpu.sync_copy(wbuf, w_hbm.at[pl.ds(base, CHUNK)])
        return carry
    lax.fori_loop(0, NCHUNK, _chunk, 0)

    plsc.subcore_barrier()
    for i in range(RPT // ZROW):
        sl = pl.ds(row0 + i * ZROW, ZROW)
        pltpu.sync_copy(den_acc.at[sl],
                        den_hbm.at[pl.ds(cid * NPAD + row0 + i * ZROW, ZROW)])


def _make_msg_kernel(num_pass, vreg_heads):

    @functools.partial(
        pl.kernel,
        mesh=_mesh,
        compiler_params=pltpu.CompilerParams(needs_layout_passes=False),
        out_type=tuple(
            jax.ShapeDtypeStruct((NC * NPAD, 128), jnp.float32)
            for _ in range(num_pass)
        ),
        scratch_types=(
            pltpu.VMEM((CHUNK,), jnp.int32),
            pltpu.VMEM((CHUNK,), jnp.int32),
            pltpu.VMEM((CHUNK, 16), jnp.float32),
            pltpu.VMEM((CHUNK, 128), jnp.float32),
            pltpu.VMEM_SHARED((NPAD, 128), jnp.float32),
            pltpu.SemaphoreType.DMA,
        ),
    )
    def msg_kernel(*refs):
        hms = refs[:num_pass]
        src_hbm, dst_hbm, w_hbm, zeros_hbm = refs[num_pass:num_pass + 4]
        outs = refs[num_pass + 4:2 * num_pass + 4]
        sidx, didx, wbuf, rbuf, acc, sem = refs[2 * num_pass + 4:]

        cid = lax.axis_index("c")
        sid = lax.axis_index("s")
        tile_base = (cid * NS + sid) * EPT
        row0 = sid * RPT

        for p in range(num_pass):
            for i in range(RPT // ZROW):
                pltpu.sync_copy(zeros_hbm, acc.at[pl.ds(row0 + i * ZROW, ZROW)])
            plsc.subcore_barrier()

            def _chunk(ci, carry):
                base = tile_base + ci * CHUNK
                pltpu.sync_copy(src_hbm.at[pl.ds(base, CHUNK)], sidx)
                pltpu.sync_copy(dst_hbm.at[pl.ds(base, CHUNK)], didx)
                pltpu.sync_copy(w_hbm.at[pl.ds(base, CHUNK)], wbuf)
                pltpu.async_copy(hms[p].at[sidx], rbuf, sem).wait()

                def _edge(e, c2):
                    for h in sorted({hh for _, hh in vreg_heads(p)}):
                        wsp = plsc.load_gather(wbuf, [_splat(e), _splat(h)])
                        for j, hj in vreg_heads(p):
                            if hj == h:
                                sl = pl.ds(16 * j, 16)
                                rbuf[e, sl] = rbuf[e, sl] * wsp
                    return c2
                lax.fori_loop(0, CHUNK, _edge, 0)

                pltpu.sync_copy(rbuf, acc.at[didx], add=True)
                return carry
            lax.fori_loop(0, NCHUNK, _chunk, 0)

            plsc.subcore_barrier()
            for i in range(RPT // ZROW):
                sl = pl.ds(row0 + i * ZROW, ZROW)
                pltpu.sync_copy(acc.at[sl],
                                outs[p].at[pl.ds(cid * NPAD + row0 + i * ZROW, ZROW)])
            plsc.subcore_barrier()

    return msg_kernel


_msg_kernel_l1 = _make_msg_kernel(
    4, lambda p: [(j, 2 * p + (j // 4)) for j in range(8)])
_msg_kernel_l2 = _make_msg_kernel(
    2, lambda p: [(j, j) for j in range(8)] if p == 0 else [(0, 8), (1, 9)])


BN = 1000
GRID = N // BN


def _node1_body(x_ref, w_ref, asv_ref, adv_ref, h_ref, asad_ref, wl_ref):
    h = jnp.dot(x_ref[...], w_ref[...], preferred_element_type=jnp.float32)
    h_ref[...] = h
    hr = h.reshape(BN, H1, C1)
    a_s = jnp.sum(hr * asv_ref[...], axis=-1)
    a_d = jnp.sum(hr * adv_ref[...], axis=-1)
    z8 = jnp.zeros((BN, 8), jnp.float32)
    asad_ref[...] = jnp.concatenate(
        [a_s, z8, a_d, jnp.zeros((BN, 104), jnp.float32)], axis=1)
    x = a_s + a_d
    x = jnp.where(x >= 0.0, x, 0.2 * x)
    wl_ref[...] = jnp.exp(x)


def _node1(x, W1, a1s, a1d):
    return pl.pallas_call(
        _node1_body,
        grid=(GRID,),
        in_specs=[
            pl.BlockSpec((BN, D), lambda i: (i, 0)),
            pl.BlockSpec((D, H1 * C1), lambda i: (0, 0)),
            pl.BlockSpec((H1, C1), lambda i: (0, 0)),
            pl.BlockSpec((H1, C1), lambda i: (0, 0)),
        ],
        out_specs=[
            pl.BlockSpec((BN, H1 * C1), lambda i: (i, 0)),
            pl.BlockSpec((BN, 128), lambda i: (i, 0)),
            pl.BlockSpec((BN, H1), lambda i: (i, 0)),
        ],
        out_shape=[
            jax.ShapeDtypeStruct((N, H1 * C1), jnp.float32),
            jax.ShapeDtypeStruct((N, 128), jnp.float32),
            jax.ShapeDtypeStruct((N, H1), jnp.float32),
        ],
    )(x, W1, a1s, a1d)


def _combine1_body(h_ref, wl_ref, d0_ref, d1_ref, b_ref,
                   a0_ref, a1_ref, a2_ref, a3_ref,
                   a4_ref, a5_ref, a6_ref, a7_ref,
                   h1_ref, he_ref):
    h = h_ref[...].reshape(BN, H1, C1)
    wl = wl_ref[...]
    den = d0_ref[...][:, :H1] + d1_ref[...][:, :H1] + wl
    pairs = [(a0_ref, a1_ref), (a2_ref, a3_ref), (a4_ref, a5_ref), (a6_ref, a7_ref)]
    acc = jnp.concatenate(
        [(p0[...] + p1[...]).reshape(BN, 2, C1) for p0, p1 in pairs], axis=1)
    out = (acc + wl[..., None] * h) / (den[..., None] + 1e-16)
    h1 = out.reshape(BN, H1 * C1) + b_ref[...]
    h1_ref[...] = h1
    he_ref[...] = jnp.where(h1 > 0.0, h1, jnp.exp(jnp.minimum(h1, 0.0)) - 1.0)


def _combine1(h, wl, den0, den1, b1, accs):
    ins = [h, wl, den0, den1, b1.reshape(1, H1 * C1)]
    for a in accs:
        ins.extend([a[:N], a[NPAD:NPAD + N]])
    return pl.pallas_call(
        _combine1_body,
        grid=(GRID,),
        in_specs=[
            pl.BlockSpec((BN, H1 * C1), lambda i: (i, 0)),
            pl.BlockSpec((BN, H1), lambda i: (i, 0)),
            pl.BlockSpec((BN, 128), lambda i: (i, 0)),
            pl.BlockSpec((BN, 128), lambda i: (i, 0)),
            pl.BlockSpec((1, H1 * C1), lambda i: (0, 0)),
        ] + [pl.BlockSpec((BN, 128), lambda i: (i, 0))] * 8,
        out_specs=[
            pl.BlockSpec((BN, H1 * C1), lambda i: (i, 0)),
            pl.BlockSpec((BN, H1 * C1), lambda i: (i, 0)),
        ],
        out_shape=[
            jax.ShapeDtypeStruct((N, H1 * C1), jnp.float32),
            jax.ShapeDtypeStruct((N, H1 * C1), jnp.float32),
        ],
    )(*ins)


def _node2_body(x_ref, w_ref, asv_ref, adv_ref,
                hma_ref, hmb_ref, asad_ref, wl_ref):
    h = jnp.dot(x_ref[...], w_ref[...], preferred_element_type=jnp.float32)
    hr = h.reshape(BN, H2, C2)
    a_s = jnp.sum(hr * asv_ref[...], axis=-1)
    a_d = jnp.sum(hr * adv_ref[...], axis=-1)
    asad_ref[...] = jnp.concatenate(
        [a_s, jnp.zeros((BN, 6), jnp.float32),
         a_d, jnp.zeros((BN, 102), jnp.float32)], axis=1)
    x = a_s + a_d
    x = jnp.where(x >= 0.0, x, 0.2 * x)
    wl_ref[...] = jnp.concatenate(
        [jnp.exp(x), jnp.zeros((BN, 16 - H2), jnp.float32)], axis=1)
    cpad = jnp.zeros((BN, H2, 16 - C2), jnp.float32)
    hp = jnp.concatenate([hr, cpad], axis=2)
    hma_ref[...] = hp[:, :8, :].reshape(BN, 128)
    hmb_ref[...] = jnp.concatenate(
        [hp[:, 8:, :].reshape(BN, 32), jnp.zeros((BN, 96), jnp.float32)],
        axis=1)


def _node2(x, W2, a2s, a2d):
    return pl.pallas_call(
        _node2_body,
        grid=(GRID,),
        in_specs=[
            pl.BlockSpec((BN, H1 * C1), lambda i: (i, 0)),
            pl.BlockSpec((H1 * C1, H2 * C2), lambda i: (0, 0)),
            pl.BlockSpec((H2, C2), lambda i: (0, 0)),
            pl.BlockSpec((H2, C2), lambda i: (0, 0)),
        ],
        out_specs=[
            pl.BlockSpec((BN, 128), lambda i: (i, 0)),
            pl.BlockSpec((BN, 128), lambda i: (i, 0)),
            pl.BlockSpec((BN, 128), lambda i: (i, 0)),
            pl.BlockSpec((BN, 16), lambda i: (i, 0)),
        ],
        out_shape=[
            jax.ShapeDtypeStruct((N, 128), jnp.float32),
            jax.ShapeDtypeStruct((N, 128), jnp.float32),
            jax.ShapeDtypeStruct((N, 128), jnp.float32),
            jax.ShapeDtypeStruct((N, 16), jnp.float32),
        ],
    )(x, W2, a2s, a2d)


def _combine2_body(hma_ref, hmb_ref, wl_ref, d0_ref, d1_ref, b_ref,
                   pa0_ref, pa1_ref, pb0_ref, pb1_ref, h2_ref):
    hm = jnp.concatenate(
        [hma_ref[...].reshape(BN, 8, 16),
         hmb_ref[...][:, :32].reshape(BN, 2, 16)], axis=1)
    wl = wl_ref[...][:, :H2]
    den = d0_ref[...][:, :H2] + d1_ref[...][:, :H2] + wl
    acc = jnp.concatenate(
        [(pa0_ref[...] + pa1_ref[...]).reshape(BN, 8, 16),
         (pb0_ref[...] + pb1_ref[...])[:, :32].reshape(BN, 2, 16)], axis=1)
    out = (acc + wl[..., None] * hm) / (den[..., None] + 1e-16)
    h2_ref[...] = jnp.mean(out[:, :, :C2], axis=1) + b_ref[...]


def _combine2(hma, hmb, wl, den0, den1, b2, acca, accb):
    return pl.pallas_call(
        _combine2_body,
        grid=(GRID,),
        in_specs=[
            pl.BlockSpec((BN, 128), lambda i: (i, 0)),
            pl.BlockSpec((BN, 128), lambda i: (i, 0)),
            pl.BlockSpec((BN, 16), lambda i: (i, 0)),
            pl.BlockSpec((BN, 128), lambda i: (i, 0)),
            pl.BlockSpec((BN, 128), lambda i: (i, 0)),
            pl.BlockSpec((1, C2), lambda i: (0, 0)),
        ] + [pl.BlockSpec((BN, 128), lambda i: (i, 0))] * 4,
        out_specs=pl.BlockSpec((BN, C2), lambda i: (i, 0)),
        out_shape=jax.ShapeDtypeStruct((N, C2), jnp.float32),
    )(hma, hmb, wl, den0, den1, b2.reshape(1, C2),
      acca[:N], acca[NPAD:NPAD + N], accb[:N], accb[NPAD:NPAD + N])


def _nll_body(h2_ref, y_ref, out_ref):
    h2 = h2_ref[...]
    m = jnp.max(h2, axis=-1, keepdims=True)
    lse = jnp.log(jnp.sum(jnp.exp(h2 - m), axis=-1, keepdims=True)) + m
    logp = h2 - lse
    onehot = (jax.lax.broadcasted_iota(jnp.int32, h2.shape, 1)
              == y_ref[...][:, None])
    out_ref[...] = -jnp.sum(jnp.where(onehot, logp, 0.0), axis=-1)


def kernel(street_embedding, edge_index, y, train_mask, W1, a1_src, a1_dst, b1,
           W2, a2_src, a2_dst, b2):
    src = edge_index[0].astype(jnp.int32)
    dst = edge_index[1].astype(jnp.int32)
    zpad = jnp.zeros((CHUNK, 128), jnp.float32)
    zstg = jnp.zeros((ZROW, 128), jnp.float32)

    h, asad1, wl1 = _node1(street_embedding, W1,
                           a1_src.reshape(H1, C1), a1_dst.reshape(H1, C1))
    w1, den1 = _w_kernel(asad1, src, dst, zpad, zstg)
    hmc = jnp.transpose(h.reshape(N, 4, 128), (1, 0, 2))
    accs1 = _msg_kernel_l1(hmc[0], hmc[1], hmc[2], hmc[3], src, dst, w1, zstg)
    h1, helu = _combine1(h, wl1, den1[:N], den1[NPAD:NPAD + N], b1, accs1)

    hma, hmb, asad2, wl2 = _node2(helu, W2,
                                  a2_src.reshape(H2, C2), a2_dst.reshape(H2, C2))
    w2, den2 = _w_kernel(asad2, src, dst, zpad, zstg)
    acca, accb = _msg_kernel_l2(hma, hmb, src, dst, w2, zstg)
    h2 = _combine2(hma, hmb, wl2, den2[:N], den2[NPAD:NPAD + N], b2, acca, accb)

    nll = pl.pallas_call(
        _nll_body,
        out_shape=jax.ShapeDtypeStruct((N,), jnp.float32),
    )(h2, y.astype(jnp.int32))
    m = train_mask.astype(jnp.float32)
    loss_su = (nll * m).sum() / jnp.maximum(m.sum(), 1.0)
    return (loss_su, h1, h2)


# speedup vs baseline: 23.2551x; 1.3630x over previous
import functools

import jax
import jax.numpy as jnp
from jax import lax
from jax.experimental import pallas as pl
from jax.experimental.pallas import tpu as pltpu
from jax.experimental.pallas import tpu_sc as plsc

N = 10000
E = 320000
D = 128
H1, C1 = 8, 64
H2, C2 = 10, 10

NC, NS = 2, 16
NT = NC * NS
EPT = E // NT
CHUNK = 80
NCHUNK = EPT // CHUNK
NPAD = 10240
RPT = NPAD // NS
ZROW = 128

_mesh = plsc.VectorSubcoreMesh(core_axis_name="c", subcore_axis_name="s")


def _splat(v):
    return jnp.full((16,), v, jnp.int32)


_LANE = None


@functools.partial(
    pl.kernel,
    mesh=_mesh,
    compiler_params=pltpu.CompilerParams(needs_layout_passes=False),
    out_type=(
        jax.ShapeDtypeStruct((E, 16), jnp.float32),
        jax.ShapeDtypeStruct((NC * NPAD, 128), jnp.float32),
    ),
    scratch_types=(
        pltpu.VMEM((CHUNK,), jnp.int32),
        pltpu.VMEM((CHUNK,), jnp.int32),
        pltpu.VMEM((CHUNK, 128), jnp.float32),
        pltpu.VMEM((CHUNK, 128), jnp.float32),
        pltpu.VMEM((CHUNK, 16), jnp.float32),
        pltpu.VMEM((CHUNK, 128), jnp.float32),
        pltpu.VMEM_SHARED((NPAD, 128), jnp.float32),
        pltpu.SemaphoreType.DMA,
    ),
)
def _w_kernel(asad_hbm, src_hbm, dst_hbm, zpad_hbm, zstg_hbm,
              w_hbm, den_hbm,
              sidx, didx, abuf, bbuf, wbuf, wpad, den_acc, sem):
    cid = lax.axis_index("c")
    sid = lax.axis_index("s")
    tile_base = (cid * NS + sid) * EPT
    row0 = sid * RPT

    pltpu.sync_copy(zpad_hbm, wpad)
    for i in range(RPT // ZROW):
        pltpu.sync_copy(zstg_hbm, den_acc.at[pl.ds(row0 + i * ZROW, ZROW)])
    plsc.subcore_barrier()

    def _chunk(ci, carry):
        base = tile_base + ci * CHUNK
        pltpu.sync_copy(src_hbm.at[pl.ds(base, CHUNK)], sidx)
        pltpu.sync_copy(dst_hbm.at[pl.ds(base, CHUNK)], didx)
        pltpu.async_copy(asad_hbm.at[sidx], abuf, sem).wait()
        pltpu.async_copy(asad_hbm.at[didx], bbuf, sem).wait()

        def _edge(e, c2):
            x = abuf[e, pl.ds(0, 16)] + bbuf[e, pl.ds(16, 16)]
            x = jnp.where(x >= 0.0, x, 0.2 * x)
            w16 = jnp.exp(x)
            wbuf[e, pl.ds(0, 16)] = w16
            wpad[e, pl.ds(0, 16)] = w16
            return c2
        lax.fori_loop(0, CHUNK, _edge, 0)

        pltpu.sync_copy(wpad, den_acc.at[didx], add=True)
        pltpu.sync_copy(wbuf, w_hbm.at[pl.ds(base, CHUNK)])
        return carry
    lax.fori_loop(0, NCHUNK, _chunk, 0)

    plsc.subcore_barrier()
    for i in range(RPT // ZROW):
        sl = pl.ds(row0 + i * ZROW, ZROW)
        pltpu.sync_copy(den_acc.at[sl],
                        den_hbm.at[pl.ds(cid * NPAD + row0 + i * ZROW, ZROW)])


def _make_msg_kernel(num_pass, vreg_heads):

    @functools.partial(
        pl.kernel,
        mesh=_mesh,
        compiler_params=pltpu.CompilerParams(needs_layout_passes=False),
        out_type=tuple(
            jax.ShapeDtypeStruct((NC * NPAD, 128), jnp.float32)
            for _ in range(num_pass)
        ),
        scratch_types=(
            pltpu.VMEM((CHUNK,), jnp.int32),
            pltpu.VMEM((CHUNK,), jnp.int32),
            pltpu.VMEM((CHUNK, 16), jnp.float32),
            pltpu.VMEM((CHUNK, 128), jnp.float32),
            pltpu.VMEM((CHUNK,), jnp.int32),
            pltpu.VMEM((CHUNK,), jnp.int32),
            pltpu.VMEM((CHUNK, 16), jnp.float32),
            pltpu.VMEM((CHUNK, 128), jnp.float32),
            pltpu.VMEM_SHARED((NPAD, 128), jnp.float32),
            pltpu.SemaphoreType.DMA,
            pltpu.SemaphoreType.DMA,
            pltpu.SemaphoreType.DMA,
            pltpu.SemaphoreType.DMA,
        ),
    )
    def msg_kernel(*refs):
        hms = refs[:num_pass]
        src_hbm, dst_hbm, w_hbm, zeros_hbm = refs[num_pass:num_pass + 4]
        outs = refs[num_pass + 4:2 * num_pass + 4]
        (sidx0, didx0, wbuf0, rbuf0, sidx1, didx1, wbuf1, rbuf1,
         acc, semi0, semi1, semg0, semg1) = refs[2 * num_pass + 4:]
        slots = ((sidx0, didx0, wbuf0, rbuf0, semi0, semg0),
                 (sidx1, didx1, wbuf1, rbuf1, semi1, semg1))

        cid = lax.axis_index("c")
        sid = lax.axis_index("s")
        tile_base = (cid * NS + sid) * EPT
        row0 = sid * RPT

        def load_idx(c, sl):
            sidx, didx, wbuf, _, semi, _ = sl
            base = jnp.minimum(tile_base + c * CHUNK, E - CHUNK)
            pltpu.async_copy(src_hbm.at[pl.ds(base, CHUNK)], sidx, semi)
            pltpu.async_copy(dst_hbm.at[pl.ds(base, CHUNK)], didx, semi)
            pltpu.async_copy(w_hbm.at[pl.ds(base, CHUNK)], wbuf, semi)

        def wait_idx(sl):
            sidx, didx, wbuf, _, semi, _ = sl
            pltpu.make_async_copy(src_hbm.at[pl.ds(0, CHUNK)], sidx, semi).wait()
            pltpu.make_async_copy(dst_hbm.at[pl.ds(0, CHUNK)], didx, semi).wait()
            pltpu.make_async_copy(w_hbm.at[pl.ds(0, CHUNK)], wbuf, semi).wait()

        def gather(p, sl):
            sidx, _, _, rbuf, _, semg = sl
            pltpu.async_copy(hms[p].at[sidx], rbuf, semg)

        def wait_gather(p, sl):
            sidx, _, _, rbuf, _, semg = sl
            pltpu.make_async_copy(hms[p].at[sidx], rbuf, semg).wait()

        def compute_scatter(p, sl):
            _, didx, wbuf, rbuf, _, _ = sl

            def _edge(e, c2):
                for h in sorted({hh for _, hh in vreg_heads(p)}):
                    wsp = plsc.load_gather(wbuf, [_splat(e), _splat(h)])
                    for j, hj in vreg_heads(p):
                        if hj == h:
                            sl2 = pl.ds(16 * j, 16)
                            rbuf[e, sl2] = rbuf[e, sl2] * wsp
                return c2
            lax.fori_loop(0, CHUNK, _edge, 0)
            pltpu.sync_copy(rbuf, acc.at[didx], add=True)

        for p in range(num_pass):
            for i in range(RPT // ZROW):
                pltpu.sync_copy(zeros_hbm, acc.at[pl.ds(row0 + i * ZROW, ZROW)])
            plsc.subcore_barrier()

            load_idx(0, slots[0])
            wait_idx(slots[0])
            gather(p, slots[0])
            load_idx(1, slots[1])

            def _pair(k, carry):
                c0 = 2 * k
                wait_idx(slots[1])
                gather(p, slots[1])
                wait_gather(p, slots[0])
                compute_scatter(p, slots[0])
                load_idx(c0 + 2, slots[0])
                wait_idx(slots[0])
                gather(p, slots[0])
                wait_gather(p, slots[1])
                compute_scatter(p, slots[1])
                load_idx(c0 + 3, slots[1])
                return carry
            lax.fori_loop(0, (NCHUNK - 1) // 2, _pair, 0)

            wait_gather(p, slots[0])
            compute_scatter(p, slots[0])
            wait_idx(slots[1])

            plsc.subcore_barrier()
            for i in range(RPT // ZROW):
                sl = pl.ds(row0 + i * ZROW, ZROW)
                pltpu.sync_copy(acc.at[sl],
                                outs[p].at[pl.ds(cid * NPAD + row0 + i * ZROW, ZROW)])
            plsc.subcore_barrier()

    return msg_kernel


_msg_kernel_l1 = _make_msg_kernel(
    4, lambda p: [(j, 2 * p + (j // 4)) for j in range(8)])
_msg_kernel_l2 = _make_msg_kernel(
    2, lambda p: [(j, j) for j in range(8)] if p == 0 else [(0, 8), (1, 9)])


BN = 1000
GRID = N // BN


def _node1_body(x_ref, w_ref, asv_ref, adv_ref, h_ref, asad_ref, wl_ref):
    h = jnp.dot(x_ref[...], w_ref[...], preferred_element_type=jnp.float32)
    h_ref[...] = h
    hr = h.reshape(BN, H1, C1)
    a_s = jnp.sum(hr * asv_ref[...], axis=-1)
    a_d = jnp.sum(hr * adv_ref[...], axis=-1)
    z8 = jnp.zeros((BN, 8), jnp.float32)
    asad_ref[...] = jnp.concatenate(
        [a_s, z8, a_d, jnp.zeros((BN, 104), jnp.float32)], axis=1)
    x = a_s + a_d
    x = jnp.where(x >= 0.0, x, 0.2 * x)
    wl_ref[...] = jnp.exp(x)


def _node1(x, W1, a1s, a1d):
    return pl.pallas_call(
        _node1_body,
        grid=(GRID,),
        in_specs=[
            pl.BlockSpec((BN, D), lambda i: (i, 0)),
            pl.BlockSpec((D, H1 * C1), lambda i: (0, 0)),
            pl.BlockSpec((H1, C1), lambda i: (0, 0)),
            pl.BlockSpec((H1, C1), lambda i: (0, 0)),
        ],
        out_specs=[
            pl.BlockSpec((BN, H1 * C1), lambda i: (i, 0)),
            pl.BlockSpec((BN, 128), lambda i: (i, 0)),
            pl.BlockSpec((BN, H1), lambda i: (i, 0)),
        ],
        out_shape=[
            jax.ShapeDtypeStruct((N, H1 * C1), jnp.float32),
            jax.ShapeDtypeStruct((N, 128), jnp.float32),
            jax.ShapeDtypeStruct((N, H1), jnp.float32),
        ],
    )(x, W1, a1s, a1d)


def _combine1_body(h_ref, wl_ref, d0_ref, d1_ref, b_ref,
                   a0_ref, a1_ref, a2_ref, a3_ref,
                   a4_ref, a5_ref, a6_ref, a7_ref,
                   h1_ref, he_ref):
    h = h_ref[...].reshape(BN, H1, C1)
    wl = wl_ref[...]
    den = d0_ref[...][:, :H1] + d1_ref[...][:, :H1] + wl
    pairs = [(a0_ref, a1_ref), (a2_ref, a3_ref), (a4_ref, a5_ref), (a6_ref, a7_ref)]
    acc = jnp.concatenate(
        [(p0[...] + p1[...]).reshape(BN, 2, C1) for p0, p1 in pairs], axis=1)
    out = (acc + wl[..., None] * h) / (den[..., None] + 1e-16)
    h1 = out.reshape(BN, H1 * C1) + b_ref[...]
    h1_ref[...] = h1
    he_ref[...] = jnp.where(h1 > 0.0, h1, jnp.exp(jnp.minimum(h1, 0.0)) - 1.0)


def _combine1(h, wl, den0, den1, b1, accs):
    ins = [h, wl, den0, den1, b1.reshape(1, H1 * C1)]
    for a in accs:
        ins.extend([a[:N], a[NPAD:NPAD + N]])
    return pl.pallas_call(
        _combine1_body,
        grid=(GRID,),
        in_specs=[
            pl.BlockSpec((BN, H1 * C1), lambda i: (i, 0)),
            pl.BlockSpec((BN, H1), lambda i: (i, 0)),
            pl.BlockSpec((BN, 128), lambda i: (i, 0)),
            pl.BlockSpec((BN, 128), lambda i: (i, 0)),
            pl.BlockSpec((1, H1 * C1), lambda i: (0, 0)),
        ] + [pl.BlockSpec((BN, 128), lambda i: (i, 0))] * 8,
        out_specs=[
            pl.BlockSpec((BN, H1 * C1), lambda i: (i, 0)),
            pl.BlockSpec((BN, H1 * C1), lambda i: (i, 0)),
        ],
        out_shape=[
            jax.ShapeDtypeStruct((N, H1 * C1), jnp.float32),
            jax.ShapeDtypeStruct((N, H1 * C1), jnp.float32),
        ],
    )(*ins)


def _node2_body(x_ref, w_ref, asv_ref, adv_ref,
                hma_ref, hmb_ref, asad_ref, wl_ref):
    h = jnp.dot(x_ref[...], w_ref[...], preferred_element_type=jnp.float32)
    hr = h.reshape(BN, H2, C2)
    a_s = jnp.sum(hr * asv_ref[...], axis=-1)
    a_d = jnp.sum(hr * adv_ref[...], axis=-1)
    asad_ref[...] = jnp.concatenate(
        [a_s, jnp.zeros((BN, 6), jnp.float32),
         a_d, jnp.zeros((BN, 102), jnp.float32)], axis=1)
    x = a_s + a_d
    x = jnp.where(x >= 0.0, x, 0.2 * x)
    wl_ref[...] = jnp.concatenate(
        [jnp.exp(x), jnp.zeros((BN, 16 - H2), jnp.float32)], axis=1)
    cpad = jnp.zeros((BN, H2, 16 - C2), jnp.float32)
    hp = jnp.concatenate([hr, cpad], axis=2)
    hma_ref[...] = hp[:, :8, :].reshape(BN, 128)
    hmb_ref[...] = jnp.concatenate(
        [hp[:, 8:, :].reshape(BN, 32), jnp.zeros((BN, 96), jnp.float32)],
        axis=1)


def _node2(x, W2, a2s, a2d):
    return pl.pallas_call(
        _node2_body,
        grid=(GRID,),
        in_specs=[
            pl.BlockSpec((BN, H1 * C1), lambda i: (i, 0)),
            pl.BlockSpec((H1 * C1, H2 * C2), lambda i: (0, 0)),
            pl.BlockSpec((H2, C2), lambda i: (0, 0)),
            pl.BlockSpec((H2, C2), lambda i: (0, 0)),
        ],
        out_specs=[
            pl.BlockSpec((BN, 128), lambda i: (i, 0)),
            pl.BlockSpec((BN, 128), lambda i: (i, 0)),
            pl.BlockSpec((BN, 128), lambda i: (i, 0)),
            pl.BlockSpec((BN, 16), lambda i: (i, 0)),
        ],
        out_shape=[
            jax.ShapeDtypeStruct((N, 128), jnp.float32),
            jax.ShapeDtypeStruct((N, 128), jnp.float32),
            jax.ShapeDtypeStruct((N, 128), jnp.float32),
            jax.ShapeDtypeStruct((N, 16), jnp.float32),
        ],
    )(x, W2, a2s, a2d)


def _combine2_body(hma_ref, hmb_ref, wl_ref, d0_ref, d1_ref, b_ref,
                   pa0_ref, pa1_ref, pb0_ref, pb1_ref, h2_ref):
    hm = jnp.concatenate(
        [hma_ref[...].reshape(BN, 8, 16),
         hmb_ref[...][:, :32].reshape(BN, 2, 16)], axis=1)
    wl = wl_ref[...][:, :H2]
    den = d0_ref[...][:, :H2] + d1_ref[...][:, :H2] + wl
    acc = jnp.concatenate(
        [(pa0_ref[...] + pa1_ref[...]).reshape(BN, 8, 16),
         (pb0_ref[...] + pb1_ref[...])[:, :32].reshape(BN, 2, 16)], axis=1)
    out = (acc + wl[..., None] * hm) / (den[..., None] + 1e-16)
    h2_ref[...] = jnp.mean(out[:, :, :C2], axis=1) + b_ref[...]


def _combine2(hma, hmb, wl, den0, den1, b2, acca, accb):
    return pl.pallas_call(
        _combine2_body,
        grid=(GRID,),
        in_specs=[
            pl.BlockSpec((BN, 128), lambda i: (i, 0)),
            pl.BlockSpec((BN, 128), lambda i: (i, 0)),
            pl.BlockSpec((BN, 16), lambda i: (i, 0)),
            pl.BlockSpec((BN, 128), lambda i: (i, 0)),
            pl.BlockSpec((BN, 128), lambda i: (i, 0)),
            pl.BlockSpec((1, C2), lambda i: (0, 0)),
        ] + [pl.BlockSpec((BN, 128), lambda i: (i, 0))] * 4,
        out_specs=pl.BlockSpec((BN, C2), lambda i: (i, 0)),
        out_shape=jax.ShapeDtypeStruct((N, C2), jnp.float32),
    )(hma, hmb, wl, den0, den1, b2.reshape(1, C2),
      acca[:N], acca[NPAD:NPAD + N], accb[:N], accb[NPAD:NPAD + N])


def _nll_body(h2_ref, y_ref, out_ref):
    h2 = h2_ref[...]
    m = jnp.max(h2, axis=-1, keepdims=True)
    lse = jnp.log(jnp.sum(jnp.exp(h2 - m), axis=-1, keepdims=True)) + m
    logp = h2 - lse
    onehot = (jax.lax.broadcasted_iota(jnp.int32, h2.shape, 1)
              == y_ref[...][:, None])
    out_ref[...] = -jnp.sum(jnp.where(onehot, logp, 0.0), axis=-1)


def kernel(street_embedding, edge_index, y, train_mask, W1, a1_src, a1_dst, b1,
           W2, a2_src, a2_dst, b2):
    src = edge_index[0].astype(jnp.int32)
    dst = edge_index[1].astype(jnp.int32)
    zpad = jnp.zeros((CHUNK, 128), jnp.float32)
    zstg = jnp.zeros((ZROW, 128), jnp.float32)

    h, asad1, wl1 = _node1(street_embedding, W1,
                           a1_src.reshape(H1, C1), a1_dst.reshape(H1, C1))
    w1, den1 = _w_kernel(asad1, src, dst, zpad, zstg)
    hmc = jnp.transpose(h.reshape(N, 4, 128), (1, 0, 2))
    accs1 = _msg_kernel_l1(hmc[0], hmc[1], hmc[2], hmc[3], src, dst, w1, zstg)
    h1, helu = _combine1(h, wl1, den1[:N], den1[NPAD:NPAD + N], b1, accs1)

    hma, hmb, asad2, wl2 = _node2(helu, W2,
                                  a2_src.reshape(H2, C2), a2_dst.reshape(H2, C2))
    w2, den2 = _w_kernel(asad2, src, dst, zpad, zstg)
    acca, accb = _msg_kernel_l2(hma, hmb, src, dst, w2, zstg)
    h2 = _combine2(hma, hmb, wl2, den2[:N], den2[NPAD:NPAD + N], b2, acca, accb)

    nll = pl.pallas_call(
        _nll_body,
        out_shape=jax.ShapeDtypeStruct((N,), jnp.float32),
    )(h2, y.astype(jnp.int32))
    m = train_mask.astype(jnp.float32)
    loss_su = (nll * m).sum() / jnp.maximum(m.sum(), 1.0)
    return (loss_su, h1, h2)


# speedup vs baseline: 26.6789x; 1.1472x over previous
import functools

import jax
import jax.numpy as jnp
from jax import lax
from jax.experimental import pallas as pl
from jax.experimental.pallas import tpu as pltpu
from jax.experimental.pallas import tpu_sc as plsc

N = 10000
E = 320000
D = 128
H1, C1 = 8, 64
H2, C2 = 10, 10

NC, NS = 2, 16
NT = NC * NS
EPT = E // NT
CHUNK = 80
NCHUNK = EPT // CHUNK
NPAD = 10240
RPT = NPAD // NS
ZROW = 128

_mesh = plsc.VectorSubcoreMesh(core_axis_name="c", subcore_axis_name="s")


def _splat(v):
    return jnp.full((16,), v, jnp.int32)


_LANE = None


CHW = 40
NCHW = EPT // CHW

@functools.partial(
    pl.kernel,
    mesh=_mesh,
    compiler_params=pltpu.CompilerParams(needs_layout_passes=False),
    out_type=(
        jax.ShapeDtypeStruct((E, 16), jnp.float32),
        jax.ShapeDtypeStruct((NC * NPAD, 128), jnp.float32),
    ),
    scratch_types=(
        pltpu.VMEM((CHW,), jnp.int32),
        pltpu.VMEM((CHW,), jnp.int32),
        pltpu.VMEM((CHW, 128), jnp.float32),
        pltpu.VMEM((CHW, 128), jnp.float32),
        pltpu.VMEM((CHW,), jnp.int32),
        pltpu.VMEM((CHW,), jnp.int32),
        pltpu.VMEM((CHW, 128), jnp.float32),
        pltpu.VMEM((CHW, 128), jnp.float32),
        pltpu.VMEM((CHW, 16), jnp.float32),
        pltpu.VMEM((CHW, 128), jnp.float32),
        pltpu.VMEM_SHARED((NPAD, 128), jnp.float32),
        pltpu.SemaphoreType.DMA,
        pltpu.SemaphoreType.DMA,
        pltpu.SemaphoreType.DMA,
        pltpu.SemaphoreType.DMA,
    ),
)
def _w_kernel(asad_hbm, src_hbm, dst_hbm, zpad_hbm, zstg_hbm,
              w_hbm, den_hbm,
              sidx0, didx0, abuf0, bbuf0, sidx1, didx1, abuf1, bbuf1,
              wbuf, wpad, den_acc, semi0, semi1, semg0, semg1):
    cid = lax.axis_index("c")
    sid = lax.axis_index("s")
    tile_base = (cid * NS + sid) * EPT
    row0 = sid * RPT
    slots = ((sidx0, didx0, abuf0, bbuf0, semi0, semg0),
             (sidx1, didx1, abuf1, bbuf1, semi1, semg1))

    def load_idx(c, sl):
        sidx, didx, _, _, semi, _ = sl
        base = jnp.minimum(tile_base + c * CHW, E - CHW)
        pltpu.async_copy(src_hbm.at[pl.ds(base, CHW)], sidx, semi)
        pltpu.async_copy(dst_hbm.at[pl.ds(base, CHW)], didx, semi)

    def wait_idx(sl):
        sidx, didx, _, _, semi, _ = sl
        pltpu.make_async_copy(src_hbm.at[pl.ds(0, CHW)], sidx, semi).wait()
        pltpu.make_async_copy(dst_hbm.at[pl.ds(0, CHW)], didx, semi).wait()

    def gather(sl):
        sidx, didx, abuf, bbuf, _, semg = sl
        pltpu.async_copy(asad_hbm.at[sidx], abuf, semg)
        pltpu.async_copy(asad_hbm.at[didx], bbuf, semg)

    def wait_gather(sl):
        sidx, didx, abuf, bbuf, _, semg = sl
        pltpu.make_async_copy(asad_hbm.at[sidx], abuf, semg).wait()
        pltpu.make_async_copy(asad_hbm.at[didx], bbuf, semg).wait()

    def compute_scatter(c, sl):
        _, didx, abuf, bbuf, _, _ = sl

        def _edge(e, c2):
            x = abuf[e, pl.ds(0, 16)] + bbuf[e, pl.ds(16, 16)]
            x = jnp.where(x >= 0.0, x, 0.2 * x)
            w16 = jnp.exp(x)
            wbuf[e, pl.ds(0, 16)] = w16
            wpad[e, pl.ds(0, 16)] = w16
            return c2
        lax.fori_loop(0, CHW, _edge, 0)
        pltpu.sync_copy(wpad, den_acc.at[didx], add=True)
        base = tile_base + c * CHW
        pltpu.sync_copy(wbuf, w_hbm.at[pl.ds(base, CHW)])

    pltpu.sync_copy(zpad_hbm, wpad)
    for i in range(RPT // ZROW):
        pltpu.sync_copy(zstg_hbm, den_acc.at[pl.ds(row0 + i * ZROW, ZROW)])
    plsc.subcore_barrier()

    load_idx(0, slots[0])
    wait_idx(slots[0])
    gather(slots[0])
    load_idx(1, slots[1])

    def _pair(k, carry):
        c0 = 2 * k
        wait_idx(slots[1])
        gather(slots[1])
        wait_gather(slots[0])
        compute_scatter(c0, slots[0])
        load_idx(c0 + 2, slots[0])
        wait_idx(slots[0])
        gather(slots[0])
        wait_gather(slots[1])
        compute_scatter(c0 + 1, slots[1])
        load_idx(c0 + 3, slots[1])
        return carry
    lax.fori_loop(0, NCHW // 2 - 1, _pair, 0)

    wait_idx(slots[1])
    gather(slots[1])
    wait_gather(slots[0])
    compute_scatter(NCHW - 2, slots[0])
    wait_gather(slots[1])
    compute_scatter(NCHW - 1, slots[1])

    plsc.subcore_barrier()
    for i in range(RPT // ZROW):
        sl = pl.ds(row0 + i * ZROW, ZROW)
        pltpu.sync_copy(den_acc.at[sl],
                        den_hbm.at[pl.ds(cid * NPAD + row0 + i * ZROW, ZROW)])


def _make_msg_kernel(num_pass, vreg_heads):

    @functools.partial(
        pl.kernel,
        mesh=_mesh,
        compiler_params=pltpu.CompilerParams(needs_layout_passes=False),
        out_type=tuple(
            jax.ShapeDtypeStruct((NC * NPAD, 128), jnp.float32)
            for _ in range(num_pass)
        ),
        scratch_types=(
            pltpu.VMEM((CHUNK,), jnp.int32),
            pltpu.VMEM((CHUNK,), jnp.int32),
            pltpu.VMEM((CHUNK, 16), jnp.float32),
            pltpu.VMEM((CHUNK, 128), jnp.float32),
            pltpu.VMEM((CHUNK,), jnp.int32),
            pltpu.VMEM((CHUNK,), jnp.int32),
            pltpu.VMEM((CHUNK, 16), jnp.float32),
            pltpu.VMEM((CHUNK, 128), jnp.float32),
            pltpu.VMEM_SHARED((NPAD, 128), jnp.float32),
            pltpu.SemaphoreType.DMA,
            pltpu.SemaphoreType.DMA,
            pltpu.SemaphoreType.DMA,
            pltpu.SemaphoreType.DMA,
        ),
    )
    def msg_kernel(*refs):
        hms = refs[:num_pass]
        src_hbm, dst_hbm, w_hbm, zeros_hbm = refs[num_pass:num_pass + 4]
        outs = refs[num_pass + 4:2 * num_pass + 4]
        (sidx0, didx0, wbuf0, rbuf0, sidx1, didx1, wbuf1, rbuf1,
         acc, semi0, semi1, semg0, semg1) = refs[2 * num_pass + 4:]
        slots = ((sidx0, didx0, wbuf0, rbuf0, semi0, semg0),
                 (sidx1, didx1, wbuf1, rbuf1, semi1, semg1))

        cid = lax.axis_index("c")
        sid = lax.axis_index("s")
        tile_base = (cid * NS + sid) * EPT
        row0 = sid * RPT

        def load_idx(c, sl):
            sidx, didx, wbuf, _, semi, _ = sl
            base = jnp.minimum(tile_base + c * CHUNK, E - CHUNK)
            pltpu.async_copy(src_hbm.at[pl.ds(base, CHUNK)], sidx, semi)
            pltpu.async_copy(dst_hbm.at[pl.ds(base, CHUNK)], didx, semi)
            pltpu.async_copy(w_hbm.at[pl.ds(base, CHUNK)], wbuf, semi)

        def wait_idx(sl):
            sidx, didx, wbuf, _, semi, _ = sl
            pltpu.make_async_copy(src_hbm.at[pl.ds(0, CHUNK)], sidx, semi).wait()
            pltpu.make_async_copy(dst_hbm.at[pl.ds(0, CHUNK)], didx, semi).wait()
            pltpu.make_async_copy(w_hbm.at[pl.ds(0, CHUNK)], wbuf, semi).wait()

        def gather(p, sl):
            sidx, _, _, rbuf, _, semg = sl
            pltpu.async_copy(hms[p].at[sidx], rbuf, semg)

        def wait_gather(p, sl):
            sidx, _, _, rbuf, _, semg = sl
            pltpu.make_async_copy(hms[p].at[sidx], rbuf, semg).wait()

        def compute_scatter(p, sl):
            _, didx, wbuf, rbuf, _, _ = sl

            def _edge(e, c2):
                for h in sorted({hh for _, hh in vreg_heads(p)}):
                    wsp = plsc.load_gather(wbuf, [_splat(e), _splat(h)])
                    for j, hj in vreg_heads(p):
                        if hj == h:
                            sl2 = pl.ds(16 * j, 16)
                            rbuf[e, sl2] = rbuf[e, sl2] * wsp
                return c2
            lax.fori_loop(0, CHUNK, _edge, 0)
            pltpu.sync_copy(rbuf, acc.at[didx], add=True)

        for p in range(num_pass):
            for i in range(RPT // ZROW):
                pltpu.sync_copy(zeros_hbm, acc.at[pl.ds(row0 + i * ZROW, ZROW)])
            plsc.subcore_barrier()

            load_idx(0, slots[0])
            wait_idx(slots[0])
            gather(p, slots[0])
            load_idx(1, slots[1])

            def _pair(k, carry):
                c0 = 2 * k
                wait_idx(slots[1])
                gather(p, slots[1])
                wait_gather(p, slots[0])
                compute_scatter(p, slots[0])
                load_idx(c0 + 2, slots[0])
                wait_idx(slots[0])
                gather(p, slots[0])
                wait_gather(p, slots[1])
                compute_scatter(p, slots[1])
                load_idx(c0 + 3, slots[1])
                return carry
            lax.fori_loop(0, (NCHUNK - 1) // 2, _pair, 0)

            wait_gather(p, slots[0])
            compute_scatter(p, slots[0])
            wait_idx(slots[1])

            plsc.subcore_barrier()
            for i in range(RPT // ZROW):
                sl = pl.ds(row0 + i * ZROW, ZROW)
                pltpu.sync_copy(acc.at[sl],
                                outs[p].at[pl.ds(cid * NPAD + row0 + i * ZROW, ZROW)])
            plsc.subcore_barrier()

    return msg_kernel


_msg_kernel_l1 = _make_msg_kernel(
    4, lambda p: [(j, 2 * p + (j // 4)) for j in range(8)])
_msg_kernel_l2 = _make_msg_kernel(
    2, lambda p: [(j, j) for j in range(8)] if p == 0 else [(0, 8), (1, 9)])


BN = 1000
GRID = N // BN


def _node1_body(x_ref, w_ref, asv_ref, adv_ref, h_ref, asad_ref, wl_ref):
    h = jnp.dot(x_ref[...], w_ref[...], preferred_element_type=jnp.float32)
    h_ref[...] = h
    hr = h.reshape(BN, H1, C1)
    a_s = jnp.sum(hr * asv_ref[...], axis=-1)
    a_d = jnp.sum(hr * adv_ref[...], axis=-1)
    z8 = jnp.zeros((BN, 8), jnp.float32)
    asad_ref[...] = jnp.concatenate(
        [a_s, z8, a_d, jnp.zeros((BN, 104), jnp.float32)], axis=1)
    x = a_s + a_d
    x = jnp.where(x >= 0.0, x, 0.2 * x)
    wl_ref[...] = jnp.exp(x)


def _node1(x, W1, a1s, a1d):
    return pl.pallas_call(
        _node1_body,
        grid=(GRID,),
        in_specs=[
            pl.BlockSpec((BN, D), lambda i: (i, 0)),
            pl.BlockSpec((D, H1 * C1), lambda i: (0, 0)),
            pl.BlockSpec((H1, C1), lambda i: (0, 0)),
            pl.BlockSpec((H1, C1), lambda i: (0, 0)),
        ],
        out_specs=[
            pl.BlockSpec((BN, H1 * C1), lambda i: (i, 0)),
            pl.BlockSpec((BN, 128), lambda i: (i, 0)),
            pl.BlockSpec((BN, H1), lambda i: (i, 0)),
        ],
        out_shape=[
            jax.ShapeDtypeStruct((N, H1 * C1), jnp.float32),
            jax.ShapeDtypeStruct((N, 128), jnp.float32),
            jax.ShapeDtypeStruct((N, H1), jnp.float32),
        ],
    )(x, W1, a1s, a1d)


def _combine1_body(h_ref, wl_ref, d0_ref, d1_ref, b_ref,
                   a0_ref, a1_ref, a2_ref, a3_ref,
                   a4_ref, a5_ref, a6_ref, a7_ref,
                   h1_ref, he_ref):
    h = h_ref[...].reshape(BN, H1, C1)
    wl = wl_ref[...]
    den = d0_ref[...][:, :H1] + d1_ref[...][:, :H1] + wl
    pairs = [(a0_ref, a1_ref), (a2_ref, a3_ref), (a4_ref, a5_ref), (a6_ref, a7_ref)]
    acc = jnp.concatenate(
        [(p0[...] + p1[...]).reshape(BN, 2, C1) for p0, p1 in pairs], axis=1)
    out = (acc + wl[..., None] * h) / (den[..., None] + 1e-16)
    h1 = out.reshape(BN, H1 * C1) + b_ref[...]
    h1_ref[...] = h1
    he_ref[...] = jnp.where(h1 > 0.0, h1, jnp.exp(jnp.minimum(h1, 0.0)) - 1.0)


def _combine1(h, wl, den0, den1, b1, accs):
    ins = [h, wl, den0, den1, b1.reshape(1, H1 * C1)]
    for a in accs:
        ins.extend([a[:N], a[NPAD:NPAD + N]])
    return pl.pallas_call(
        _combine1_body,
        grid=(GRID,),
        in_specs=[
            pl.BlockSpec((BN, H1 * C1), lambda i: (i, 0)),
            pl.BlockSpec((BN, H1), lambda i: (i, 0)),
            pl.BlockSpec((BN, 128), lambda i: (i, 0)),
            pl.BlockSpec((BN, 128), lambda i: (i, 0)),
            pl.BlockSpec((1, H1 * C1), lambda i: (0, 0)),
        ] + [pl.BlockSpec((BN, 128), lambda i: (i, 0))] * 8,
        out_specs=[
            pl.BlockSpec((BN, H1 * C1), lambda i: (i, 0)),
            pl.BlockSpec((BN, H1 * C1), lambda i: (i, 0)),
        ],
        out_shape=[
            jax.ShapeDtypeStruct((N, H1 * C1), jnp.float32),
            jax.ShapeDtypeStruct((N, H1 * C1), jnp.float32),
        ],
    )(*ins)


def _node2_body(x_ref, w_ref, asv_ref, adv_ref,
                hma_ref, hmb_ref, asad_ref, wl_ref):
    h = jnp.dot(x_ref[...], w_ref[...], preferred_element_type=jnp.float32)
    hr = h.reshape(BN, H2, C2)
    a_s = jnp.sum(hr * asv_ref[...], axis=-1)
    a_d = jnp.sum(hr * adv_ref[...], axis=-1)
    asad_ref[...] = jnp.concatenate(
        [a_s, jnp.zeros((BN, 6), jnp.float32),
         a_d, jnp.zeros((BN, 102), jnp.float32)], axis=1)
    x = a_s + a_d
    x = jnp.where(x >= 0.0, x, 0.2 * x)
    wl_ref[...] = jnp.concatenate(
        [jnp.exp(x), jnp.zeros((BN, 16 - H2), jnp.float32)], axis=1)
    cpad = jnp.zeros((BN, H2, 16 - C2), jnp.float32)
    hp = jnp.concatenate([hr, cpad], axis=2)
    hma_ref[...] = hp[:, :8, :].reshape(BN, 128)
    hmb_ref[...] = jnp.concatenate(
        [hp[:, 8:, :].reshape(BN, 32), jnp.zeros((BN, 96), jnp.float32)],
        axis=1)


def _node2(x, W2, a2s, a2d):
    return pl.pallas_call(
        _node2_body,
        grid=(GRID,),
        in_specs=[
            pl.BlockSpec((BN, H1 * C1), lambda i: (i, 0)),
            pl.BlockSpec((H1 * C1, H2 * C2), lambda i: (0, 0)),
            pl.BlockSpec((H2, C2), lambda i: (0, 0)),
            pl.BlockSpec((H2, C2), lambda i: (0, 0)),
        ],
        out_specs=[
            pl.BlockSpec((BN, 128), lambda i: (i, 0)),
            pl.BlockSpec((BN, 128), lambda i: (i, 0)),
            pl.BlockSpec((BN, 128), lambda i: (i, 0)),
            pl.BlockSpec((BN, 16), lambda i: (i, 0)),
        ],
        out_shape=[
            jax.ShapeDtypeStruct((N, 128), jnp.float32),
            jax.ShapeDtypeStruct((N, 128), jnp.float32),
            jax.ShapeDtypeStruct((N, 128), jnp.float32),
            jax.ShapeDtypeStruct((N, 16), jnp.float32),
        ],
    )(x, W2, a2s, a2d)


def _combine2_body(hma_ref, hmb_ref, wl_ref, d0_ref, d1_ref, b_ref,
                   pa0_ref, pa1_ref, pb0_ref, pb1_ref, h2_ref):
    hm = jnp.concatenate(
        [hma_ref[...].reshape(BN, 8, 16),
         hmb_ref[...][:, :32].reshape(BN, 2, 16)], axis=1)
    wl = wl_ref[...][:, :H2]
    den = d0_ref[...][:, :H2] + d1_ref[...][:, :H2] + wl
    acc = jnp.concatenate(
        [(pa0_ref[...] + pa1_ref[...]).reshape(BN, 8, 16),
         (pb0_ref[...] + pb1_ref[...])[:, :32].reshape(BN, 2, 16)], axis=1)
    out = (acc + wl[..., None] * hm) / (den[..., None] + 1e-16)
    h2_ref[...] = jnp.mean(out[:, :, :C2], axis=1) + b_ref[...]


def _combine2(hma, hmb, wl, den0, den1, b2, acca, accb):
    return pl.pallas_call(
        _combine2_body,
        grid=(GRID,),
        in_specs=[
            pl.BlockSpec((BN, 128), lambda i: (i, 0)),
            pl.BlockSpec((BN, 128), lambda i: (i, 0)),
            pl.BlockSpec((BN, 16), lambda i: (i, 0)),
            pl.BlockSpec((BN, 128), lambda i: (i, 0)),
            pl.BlockSpec((BN, 128), lambda i: (i, 0)),
            pl.BlockSpec((1, C2), lambda i: (0, 0)),
        ] + [pl.BlockSpec((BN, 128), lambda i: (i, 0))] * 4,
        out_specs=pl.BlockSpec((BN, C2), lambda i: (i, 0)),
        out_shape=jax.ShapeDtypeStruct((N, C2), jnp.float32),
    )(hma, hmb, wl, den0, den1, b2.reshape(1, C2),
      acca[:N], acca[NPAD:NPAD + N], accb[:N], accb[NPAD:NPAD + N])


def _nll_body(h2_ref, y_ref, out_ref):
    h2 = h2_ref[...]
    m = jnp.max(h2, axis=-1, keepdims=True)
    lse = jnp.log(jnp.sum(jnp.exp(h2 - m), axis=-1, keepdims=True)) + m
    logp = h2 - lse
    onehot = (jax.lax.broadcasted_iota(jnp.int32, h2.shape, 1)
              == y_ref[...][:, None])
    out_ref[...] = -jnp.sum(jnp.where(onehot, logp, 0.0), axis=-1)


def kernel(street_embedding, edge_index, y, train_mask, W1, a1_src, a1_dst, b1,
           W2, a2_src, a2_dst, b2):
    src = edge_index[0].astype(jnp.int32)
    dst = edge_index[1].astype(jnp.int32)
    zpad = jnp.zeros((CHW, 128), jnp.float32)
    zstg = jnp.zeros((ZROW, 128), jnp.float32)

    h, asad1, wl1 = _node1(street_embedding, W1,
                           a1_src.reshape(H1, C1), a1_dst.reshape(H1, C1))
    w1, den1 = _w_kernel(asad1, src, dst, zpad, zstg)
    hmc = jnp.transpose(h.reshape(N, 4, 128), (1, 0, 2))
    accs1 = _msg_kernel_l1(hmc[0], hmc[1], hmc[2], hmc[3], src, dst, w1, zstg)
    h1, helu = _combine1(h, wl1, den1[:N], den1[NPAD:NPAD + N], b1, accs1)

    hma, hmb, asad2, wl2 = _node2(helu, W2,
                                  a2_src.reshape(H2, C2), a2_dst.reshape(H2, C2))
    w2, den2 = _w_kernel(asad2, src, dst, zpad, zstg)
    acca, accb = _msg_kernel_l2(hma, hmb, src, dst, w2, zstg)
    h2 = _combine2(hma, hmb, wl2, den2[:N], den2[NPAD:NPAD + N], b2, acca, accb)

    nll = pl.pallas_call(
        _nll_body,
        out_shape=jax.ShapeDtypeStruct((N,), jnp.float32),
    )(h2, y.astype(jnp.int32))
    m = train_mask.astype(jnp.float32)
    loss_su = (nll * m).sum() / jnp.maximum(m.sum(), 1.0)
    return (loss_su, h1, h2)


# speedup vs baseline: 27.2317x; 1.0207x over previous
import functools

import jax
import jax.numpy as jnp
from jax import lax
from jax.experimental import pallas as pl
from jax.experimental.pallas import tpu as pltpu
from jax.experimental.pallas import tpu_sc as plsc

N = 10000
E = 320000
D = 128
H1, C1 = 8, 64
H2, C2 = 10, 10

NC, NS = 2, 16
NT = NC * NS
EPT = E // NT
CHUNK = 80
NCHUNK = EPT // CHUNK
NPAD = 10240
RPT = NPAD // NS
ZROW = 128

_mesh = plsc.VectorSubcoreMesh(core_axis_name="c", subcore_axis_name="s")


def _splat(v):
    return jnp.full((16,), v, jnp.int32)


_LANE = None


CHW = 40
NCHW = EPT // CHW

@functools.partial(
    pl.kernel,
    mesh=_mesh,
    compiler_params=pltpu.CompilerParams(needs_layout_passes=False),
    out_type=(
        jax.ShapeDtypeStruct((E, 16), jnp.float32),
        jax.ShapeDtypeStruct((NC * NPAD, 128), jnp.float32),
    ),
    scratch_types=(
        pltpu.VMEM((CHW,), jnp.int32),
        pltpu.VMEM((CHW,), jnp.int32),
        pltpu.VMEM((CHW, 128), jnp.float32),
        pltpu.VMEM((CHW, 128), jnp.float32),
        pltpu.VMEM((CHW,), jnp.int32),
        pltpu.VMEM((CHW,), jnp.int32),
        pltpu.VMEM((CHW, 128), jnp.float32),
        pltpu.VMEM((CHW, 128), jnp.float32),
        pltpu.VMEM((CHW, 16), jnp.float32),
        pltpu.VMEM((CHW, 128), jnp.float32),
        pltpu.VMEM_SHARED((NPAD, 128), jnp.float32),
        pltpu.SemaphoreType.DMA,
        pltpu.SemaphoreType.DMA,
        pltpu.SemaphoreType.DMA,
        pltpu.SemaphoreType.DMA,
    ),
)
def _w_kernel(asad_hbm, src_hbm, dst_hbm, zpad_hbm, zstg_hbm,
              w_hbm, den_hbm,
              sidx0, didx0, abuf0, bbuf0, sidx1, didx1, abuf1, bbuf1,
              wbuf, wpad, den_acc, semi0, semi1, semg0, semg1):
    cid = lax.axis_index("c")
    sid = lax.axis_index("s")
    tile_base = (cid * NS + sid) * EPT
    row0 = sid * RPT
    slots = ((sidx0, didx0, abuf0, bbuf0, semi0, semg0),
             (sidx1, didx1, abuf1, bbuf1, semi1, semg1))

    def load_idx(c, sl):
        sidx, didx, _, _, semi, _ = sl
        base = jnp.minimum(tile_base + c * CHW, E - CHW)
        pltpu.async_copy(src_hbm.at[pl.ds(base, CHW)], sidx, semi)
        pltpu.async_copy(dst_hbm.at[pl.ds(base, CHW)], didx, semi)

    def wait_idx(sl):
        sidx, didx, _, _, semi, _ = sl
        pltpu.make_async_copy(src_hbm.at[pl.ds(0, CHW)], sidx, semi).wait()
        pltpu.make_async_copy(dst_hbm.at[pl.ds(0, CHW)], didx, semi).wait()

    def gather(sl):
        sidx, didx, abuf, bbuf, _, semg = sl
        pltpu.async_copy(asad_hbm.at[sidx], abuf, semg)
        pltpu.async_copy(asad_hbm.at[didx], bbuf, semg)

    def wait_gather(sl):
        sidx, didx, abuf, bbuf, _, semg = sl
        pltpu.make_async_copy(asad_hbm.at[sidx], abuf, semg).wait()
        pltpu.make_async_copy(asad_hbm.at[didx], bbuf, semg).wait()

    def compute_scatter(c, sl):
        _, didx, abuf, bbuf, _, _ = sl

        def _edge(e, c2):
            x = abuf[e, pl.ds(0, 16)] + bbuf[e, pl.ds(16, 16)]
            x = jnp.where(x >= 0.0, x, 0.2 * x)
            w16 = jnp.exp(x)
            wbuf[e, pl.ds(0, 16)] = w16
            wpad[e, pl.ds(0, 16)] = w16
            return c2
        lax.fori_loop(0, CHW, _edge, 0)
        pltpu.sync_copy(wpad, den_acc.at[didx], add=True)
        base = tile_base + c * CHW
        pltpu.sync_copy(wbuf, w_hbm.at[pl.ds(base, CHW)])

    pltpu.sync_copy(zpad_hbm, wpad)
    for i in range(RPT // ZROW):
        pltpu.sync_copy(zstg_hbm, den_acc.at[pl.ds(row0 + i * ZROW, ZROW)])
    plsc.subcore_barrier()

    load_idx(0, slots[0])
    wait_idx(slots[0])
    gather(slots[0])
    load_idx(1, slots[1])

    def _pair(k, carry):
        c0 = 2 * k
        wait_idx(slots[1])
        gather(slots[1])
        wait_gather(slots[0])
        compute_scatter(c0, slots[0])
        load_idx(c0 + 2, slots[0])
        wait_idx(slots[0])
        gather(slots[0])
        wait_gather(slots[1])
        compute_scatter(c0 + 1, slots[1])
        load_idx(c0 + 3, slots[1])
        return carry
    lax.fori_loop(0, NCHW // 2 - 1, _pair, 0)

    wait_idx(slots[1])
    gather(slots[1])
    wait_gather(slots[0])
    compute_scatter(NCHW - 2, slots[0])
    wait_gather(slots[1])
    compute_scatter(NCHW - 1, slots[1])

    plsc.subcore_barrier()
    for i in range(RPT // ZROW):
        sl = pl.ds(row0 + i * ZROW, ZROW)
        pltpu.sync_copy(den_acc.at[sl],
                        den_hbm.at[pl.ds(cid * NPAD + row0 + i * ZROW, ZROW)])


def _make_msg_kernel(num_pass, vreg_heads):

    @functools.partial(
        pl.kernel,
        mesh=_mesh,
        compiler_params=pltpu.CompilerParams(needs_layout_passes=False),
        out_type=tuple(
            jax.ShapeDtypeStruct((NC * NPAD, 128), jnp.float32)
            for _ in range(num_pass)
        ),
        scratch_types=(
            pltpu.VMEM((CHUNK,), jnp.int32),
            pltpu.VMEM((CHUNK,), jnp.int32),
            pltpu.VMEM((CHUNK,), jnp.int32),
            pltpu.VMEM((CHUNK, 16), jnp.float32),
            pltpu.VMEM((CHUNK, 128), jnp.float32),
            pltpu.VMEM((CHUNK,), jnp.int32),
            pltpu.VMEM((CHUNK,), jnp.int32),
            pltpu.VMEM((CHUNK,), jnp.int32),
            pltpu.VMEM((CHUNK, 16), jnp.float32),
            pltpu.VMEM((CHUNK, 128), jnp.float32),
            pltpu.VMEM_SHARED((NPAD, 128), jnp.float32),
            pltpu.SemaphoreType.DMA,
            pltpu.SemaphoreType.DMA,
            pltpu.SemaphoreType.DMA,
            pltpu.SemaphoreType.DMA,
            pltpu.SemaphoreType.DMA,
            pltpu.SemaphoreType.DMA,
        ),
    )
    def msg_kernel(*refs):
        hms = refs[:num_pass]
        src_hbm, dst_hbm, w_hbm, zeros_hbm = refs[num_pass:num_pass + 4]
        outs = refs[num_pass + 4:2 * num_pass + 4]
        (sidx0, didx0, dcp0, wbuf0, rbuf0, sidx1, didx1, dcp1, wbuf1, rbuf1,
         acc, semi0, semi1, semg0, semg1, sems0, sems1) = refs[2 * num_pass + 4:]
        slots = ((sidx0, didx0, dcp0, wbuf0, rbuf0, semi0, semg0, sems0),
                 (sidx1, didx1, dcp1, wbuf1, rbuf1, semi1, semg1, sems1))

        cid = lax.axis_index("c")
        sid = lax.axis_index("s")
        tile_base = (cid * NS + sid) * EPT
        row0 = sid * RPT

        def load_idx(c, sl):
            sidx, didx, _, wbuf, _, semi, _, _ = sl
            base = jnp.minimum(tile_base + c * CHUNK, E - CHUNK)
            pltpu.async_copy(src_hbm.at[pl.ds(base, CHUNK)], sidx, semi)
            pltpu.async_copy(dst_hbm.at[pl.ds(base, CHUNK)], didx, semi)
            pltpu.async_copy(w_hbm.at[pl.ds(base, CHUNK)], wbuf, semi)

        def wait_idx(sl):
            sidx, didx, _, wbuf, _, semi, _, _ = sl
            pltpu.make_async_copy(src_hbm.at[pl.ds(0, CHUNK)], sidx, semi).wait()
            pltpu.make_async_copy(dst_hbm.at[pl.ds(0, CHUNK)], didx, semi).wait()
            pltpu.make_async_copy(w_hbm.at[pl.ds(0, CHUNK)], wbuf, semi).wait()

        def gather(p, sl):
            sidx, _, _, _, rbuf, _, semg, _ = sl
            pltpu.async_copy(hms[p].at[sidx], rbuf, semg)

        def wait_gather(p, sl):
            sidx, _, _, _, rbuf, _, semg, _ = sl
            pltpu.make_async_copy(hms[p].at[sidx], rbuf, semg).wait()

        def compute(p, sl):
            _, _, _, wbuf, rbuf, _, _, _ = sl

            def _edge(e, c2):
                for h in sorted({hh for _, hh in vreg_heads(p)}):
                    wsp = plsc.load_gather(wbuf, [_splat(e), _splat(h)])
                    for j, hj in vreg_heads(p):
                        if hj == h:
                            sl2 = pl.ds(16 * j, 16)
                            rbuf[e, sl2] = rbuf[e, sl2] * wsp
                return c2
            lax.fori_loop(0, CHUNK, _edge, 0)

        def scatter_async(sl):
            _, didx, dcp, _, rbuf, _, _, sems = sl
            for v in range(CHUNK // 16):
                dcp[pl.ds(16 * v, 16)] = didx[pl.ds(16 * v, 16)]
            pltpu.async_copy(rbuf, acc.at[dcp], sems, add=True)

        def wait_scatter(sl):
            _, _, dcp, _, rbuf, _, _, sems = sl
            pltpu.make_async_copy(rbuf, acc.at[dcp], sems).wait()

        for p in range(num_pass):
            for i in range(RPT // ZROW):
                pltpu.sync_copy(zeros_hbm, acc.at[pl.ds(row0 + i * ZROW, ZROW)])
            plsc.subcore_barrier()

            load_idx(0, slots[0])
            wait_idx(slots[0])
            gather(p, slots[0])
            load_idx(1, slots[1])

            def _pair(k, carry):
                c0 = 2 * k
                wait_gather(p, slots[0])
                compute(p, slots[0])
                pl.when(k != 0)(lambda: wait_scatter(slots[1]))
                scatter_async(slots[0])
                wait_idx(slots[1])
                gather(p, slots[1])
                load_idx(c0 + 2, slots[0])
                wait_gather(p, slots[1])
                compute(p, slots[1])
                wait_scatter(slots[0])
                scatter_async(slots[1])
                wait_idx(slots[0])
                gather(p, slots[0])
                load_idx(c0 + 3, slots[1])
                return carry
            lax.fori_loop(0, (NCHUNK - 1) // 2, _pair, 0)

            wait_gather(p, slots[0])
            compute(p, slots[0])
            wait_scatter(slots[1])
            scatter_async(slots[0])
            wait_scatter(slots[0])
            wait_idx(slots[1])

            plsc.subcore_barrier()
            for i in range(RPT // ZROW):
                sl = pl.ds(row0 + i * ZROW, ZROW)
                pltpu.sync_copy(acc.at[sl],
                                outs[p].at[pl.ds(cid * NPAD + row0 + i * ZROW, ZROW)])
            plsc.subcore_barrier()

    return msg_kernel


_msg_kernel_l1 = _make_msg_kernel(
    4, lambda p: [(j, 2 * p + (j // 4)) for j in range(8)])
_msg_kernel_l2 = _make_msg_kernel(
    2, lambda p: [(j, j) for j in range(8)] if p == 0 else [(0, 8), (1, 9)])


BN = 1000
GRID = N // BN


def _node1_body(x_ref, w_ref, asv_ref, adv_ref, h_ref, asad_ref, wl_ref):
    h = jnp.dot(x_ref[...], w_ref[...], preferred_element_type=jnp.float32)
    h_ref[...] = h
    hr = h.reshape(BN, H1, C1)
    a_s = jnp.sum(hr * asv_ref[...], axis=-1)
    a_d = jnp.sum(hr * adv_ref[...], axis=-1)
    z8 = jnp.zeros((BN, 8), jnp.float32)
    asad_ref[...] = jnp.concatenate(
        [a_s, z8, a_d, jnp.zeros((BN, 104), jnp.float32)], axis=1)
    x = a_s + a_d
    x = jnp.where(x >= 0.0, x, 0.2 * x)
    wl_ref[...] = jnp.exp(x)


def _node1(x, W1, a1s, a1d):
    return pl.pallas_call(
        _node1_body,
        grid=(GRID,),
        in_specs=[
            pl.BlockSpec((BN, D), lambda i: (i, 0)),
            pl.BlockSpec((D, H1 * C1), lambda i: (0, 0)),
            pl.BlockSpec((H1, C1), lambda i: (0, 0)),
            pl.BlockSpec((H1, C1), lambda i: (0, 0)),
        ],
        out_specs=[
            pl.BlockSpec((BN, H1 * C1), lambda i: (i, 0)),
            pl.BlockSpec((BN, 128), lambda i: (i, 0)),
            pl.BlockSpec((BN, H1), lambda i: (i, 0)),
        ],
        out_shape=[
            jax.ShapeDtypeStruct((N, H1 * C1), jnp.float32),
            jax.ShapeDtypeStruct((N, 128), jnp.float32),
            jax.ShapeDtypeStruct((N, H1), jnp.float32),
        ],
    )(x, W1, a1s, a1d)


def _combine1_body(h_ref, wl_ref, d0_ref, d1_ref, b_ref,
                   a0_ref, a1_ref, a2_ref, a3_ref,
                   a4_ref, a5_ref, a6_ref, a7_ref,
                   h1_ref, he_ref):
    h = h_ref[...].reshape(BN, H1, C1)
    wl = wl_ref[...]
    den = d0_ref[...][:, :H1] + d1_ref[...][:, :H1] + wl
    pairs = [(a0_ref, a1_ref), (a2_ref, a3_ref), (a4_ref, a5_ref), (a6_ref, a7_ref)]
    acc = jnp.concatenate(
        [(p0[...] + p1[...]).reshape(BN, 2, C1) for p0, p1 in pairs], axis=1)
    out = (acc + wl[..., None] * h) / (den[..., None] + 1e-16)
    h1 = out.reshape(BN, H1 * C1) + b_ref[...]
    h1_ref[...] = h1
    he_ref[...] = jnp.where(h1 > 0.0, h1, jnp.exp(jnp.minimum(h1, 0.0)) - 1.0)


def _combine1(h, wl, den0, den1, b1, accs):
    ins = [h, wl, den0, den1, b1.reshape(1, H1 * C1)]
    for a in accs:
        ins.extend([a[:N], a[NPAD:NPAD + N]])
    return pl.pallas_call(
        _combine1_body,
        grid=(GRID,),
        in_specs=[
            pl.BlockSpec((BN, H1 * C1), lambda i: (i, 0)),
            pl.BlockSpec((BN, H1), lambda i: (i, 0)),
            pl.BlockSpec((BN, 128), lambda i: (i, 0)),
            pl.BlockSpec((BN, 128), lambda i: (i, 0)),
            pl.BlockSpec((1, H1 * C1), lambda i: (0, 0)),
        ] + [pl.BlockSpec((BN, 128), lambda i: (i, 0))] * 8,
        out_specs=[
            pl.BlockSpec((BN, H1 * C1), lambda i: (i, 0)),
            pl.BlockSpec((BN, H1 * C1), lambda i: (i, 0)),
        ],
        out_shape=[
            jax.ShapeDtypeStruct((N, H1 * C1), jnp.float32),
            jax.ShapeDtypeStruct((N, H1 * C1), jnp.float32),
        ],
    )(*ins)


def _node2_body(x_ref, w_ref, asv_ref, adv_ref,
                hma_ref, hmb_ref, asad_ref, wl_ref):
    h = jnp.dot(x_ref[...], w_ref[...], preferred_element_type=jnp.float32)
    hr = h.reshape(BN, H2, C2)
    a_s = jnp.sum(hr * asv_ref[...], axis=-1)
    a_d = jnp.sum(hr * adv_ref[...], axis=-1)
    asad_ref[...] = jnp.concatenate(
        [a_s, jnp.zeros((BN, 6), jnp.float32),
         a_d, jnp.zeros((BN, 102), jnp.float32)], axis=1)
    x = a_s + a_d
    x = jnp.where(x >= 0.0, x, 0.2 * x)
    wl_ref[...] = jnp.concatenate(
        [jnp.exp(x), jnp.zeros((BN, 16 - H2), jnp.float32)], axis=1)
    cpad = jnp.zeros((BN, H2, 16 - C2), jnp.float32)
    hp = jnp.concatenate([hr, cpad], axis=2)
    hma_ref[...] = hp[:, :8, :].reshape(BN, 128)
    hmb_ref[...] = jnp.concatenate(
        [hp[:, 8:, :].reshape(BN, 32), jnp.zeros((BN, 96), jnp.float32)],
        axis=1)


def _node2(x, W2, a2s, a2d):
    return pl.pallas_call(
        _node2_body,
        grid=(GRID,),
        in_specs=[
            pl.BlockSpec((BN, H1 * C1), lambda i: (i, 0)),
            pl.BlockSpec((H1 * C1, H2 * C2), lambda i: (0, 0)),
            pl.BlockSpec((H2, C2), lambda i: (0, 0)),
            pl.BlockSpec((H2, C2), lambda i: (0, 0)),
        ],
        out_specs=[
            pl.BlockSpec((BN, 128), lambda i: (i, 0)),
            pl.BlockSpec((BN, 128), lambda i: (i, 0)),
            pl.BlockSpec((BN, 128), lambda i: (i, 0)),
            pl.BlockSpec((BN, 16), lambda i: (i, 0)),
        ],
        out_shape=[
            jax.ShapeDtypeStruct((N, 128), jnp.float32),
            jax.ShapeDtypeStruct((N, 128), jnp.float32),
            jax.ShapeDtypeStruct((N, 128), jnp.float32),
            jax.ShapeDtypeStruct((N, 16), jnp.float32),
        ],
    )(x, W2, a2s, a2d)


def _combine2_body(hma_ref, hmb_ref, wl_ref, d0_ref, d1_ref, b_ref,
                   pa0_ref, pa1_ref, pb0_ref, pb1_ref, h2_ref):
    hm = jnp.concatenate(
        [hma_ref[...].reshape(BN, 8, 16),
         hmb_ref[...][:, :32].reshape(BN, 2, 16)], axis=1)
    wl = wl_ref[...][:, :H2]
    den = d0_ref[...][:, :H2] + d1_ref[...][:, :H2] + wl
    acc = jnp.concatenate(
        [(pa0_ref[...] + pa1_ref[...]).reshape(BN, 8, 16),
         (pb0_ref[...] + pb1_ref[...])[:, :32].reshape(BN, 2, 16)], axis=1)
    out = (acc + wl[..., None] * hm) / (den[..., None] + 1e-16)
    h2_ref[...] = jnp.mean(out[:, :, :C2], axis=1) + b_ref[...]


def _combine2(hma, hmb, wl, den0, den1, b2, acca, accb):
    return pl.pallas_call(
        _combine2_body,
        grid=(GRID,),
        in_specs=[
            pl.BlockSpec((BN, 128), lambda i: (i, 0)),
            pl.BlockSpec((BN, 128), lambda i: (i, 0)),
            pl.BlockSpec((BN, 16), lambda i: (i, 0)),
            pl.BlockSpec((BN, 128), lambda i: (i, 0)),
            pl.BlockSpec((BN, 128), lambda i: (i, 0)),
            pl.BlockSpec((1, C2), lambda i: (0, 0)),
        ] + [pl.BlockSpec((BN, 128), lambda i: (i, 0))] * 4,
        out_specs=pl.BlockSpec((BN, C2), lambda i: (i, 0)),
        out_shape=jax.ShapeDtypeStruct((N, C2), jnp.float32),
    )(hma, hmb, wl, den0, den1, b2.reshape(1, C2),
      acca[:N], acca[NPAD:NPAD + N], accb[:N], accb[NPAD:NPAD + N])


def _nll_body(h2_ref, y_ref, out_ref):
    h2 = h2_ref[...]
    m = jnp.max(h2, axis=-1, keepdims=True)
    lse = jnp.log(jnp.sum(jnp.exp(h2 - m), axis=-1, keepdims=True)) + m
    logp = h2 - lse
    onehot = (jax.lax.broadcasted_iota(jnp.int32, h2.shape, 1)
              == y_ref[...][:, None])
    out_ref[...] = -jnp.sum(jnp.where(onehot, logp, 0.0), axis=-1)


def kernel(street_embedding, edge_index, y, train_mask, W1, a1_src, a1_dst, b1,
           W2, a2_src, a2_dst, b2):
    src = edge_index[0].astype(jnp.int32)
    dst = edge_index[1].astype(jnp.int32)
    zpad = jnp.zeros((CHW, 128), jnp.float32)
    zstg = jnp.zeros((ZROW, 128), jnp.float32)

    h, asad1, wl1 = _node1(street_embedding, W1,
                           a1_src.reshape(H1, C1), a1_dst.reshape(H1, C1))
    w1, den1 = _w_kernel(asad1, src, dst, zpad, zstg)
    hmc = jnp.transpose(h.reshape(N, 4, 128), (1, 0, 2))
    accs1 = _msg_kernel_l1(hmc[0], hmc[1], hmc[2], hmc[3], src, dst, w1, zstg)
    h1, helu = _combine1(h, wl1, den1[:N], den1[NPAD:NPAD + N], b1, accs1)

    hma, hmb, asad2, wl2 = _node2(helu, W2,
                                  a2_src.reshape(H2, C2), a2_dst.reshape(H2, C2))
    w2, den2 = _w_kernel(asad2, src, dst, zpad, zstg)
    acca, accb = _msg_kernel_l2(hma, hmb, src, dst, w2, zstg)
    h2 = _combine2(hma, hmb, wl2, den2[:N], den2[NPAD:NPAD + N], b2, acca, accb)

    nll = pl.pallas_call(
        _nll_body,
        out_shape=jax.ShapeDtypeStruct((N,), jnp.float32),
    )(h2, y.astype(jnp.int32))
    m = train_mask.astype(jnp.float32)
    loss_su = (nll * m).sum() / jnp.maximum(m.sum(), 1.0)
    return (loss_su, h1, h2)
